# baseline (speedup 1.0000x reference)
import functools

import jax
import jax.numpy as jnp
import numpy as np
from jax.experimental import pallas as pl


_NB = 4


def _im2col(flat, C, khalf, qmod):
    pad = 32 * khalf + khalf
    fp = jnp.pad(flat, ((pad, pad), (0, 0)))
    pieces = []
    for dy in range(-khalf, khalf + 1):
        for dx in range(-khalf, khalf + 1):
            off = pad + 32 * dy + dx
            p = jax.lax.slice(fp, (off, 0), (off + 1024, C))
            if dx != 0:
                valid = (qmod + dx >= 0) & (qmod + dx < 32)
                p = jnp.where(valid, p, 0.0)
            pieces.append(p)
    return jnp.concatenate(pieces, axis=1)


def _fused_kernel(xflat_ref, xp_ref,
                  w1_ref, b1_ref, w2_ref, b2_ref, w3_ref, b3_ref,
                  fcw_ref, fcb_ref,
                  wh_ref, bh_ref, ah_ref,
                  ws_ref, bs_ref, as_ref,
                  wm_ref, bm_ref, am_ref,
                  we_ref, be_ref, ae_ref,
                  wt_ref, bt_ref,
                  out_ref):
    nb = xflat_ref.shape[0]

    xp = xp_ref[...].reshape(nb * 64, 48)
    h = xp @ w1_ref[...] + b1_ref[...]
    h = jnp.where(h > 0, h, 0.1 * h)
    h = h @ w2_ref[...] + b2_ref[...]
    h = jnp.where(h > 0, h, 0.1 * h)
    h = h @ w3_ref[...] + b3_ref[...]
    h = jnp.where(h > 0, h, 0.1 * h)
    gap = jnp.mean(h.reshape(nb, 64, 128), axis=1)
    logits = gap @ fcw_ref[...] + fcb_ref[...]

    l0 = logits[:, 0:1]
    l1 = logits[:, 1:2]
    l2 = logits[:, 2:3]
    e0 = (l0 >= l1) & (l0 >= l2)
    e1 = jnp.logical_not(e0) & (l1 >= l2)
    e2 = jnp.logical_not(e0) & jnp.logical_not(e1)
    coeff = jnp.concatenate(
        [e0.astype(jnp.float32), e1.astype(jnp.float32), e2.astype(jnp.float32)],
        axis=1)

    qmod = jax.lax.broadcasted_iota(jnp.int32, (1024, 1), 0) % 32

    for i in range(nb):
        c0 = coeff[i:i + 1, 0:1]
        c1 = coeff[i:i + 1, 1:2]
        c2 = coeff[i:i + 1, 2:3]

        def blend(get):
            return c0 * get(0) + c1 * get(1) + c2 * get(2)

        p0 = _im2col(xflat_ref[i], 3, 2, qmod)
        wh = blend(lambda e: wh_ref[e])
        t = jnp.dot(p0, wh, preferred_element_type=jnp.float32)
        t = t + blend(lambda e: bh_ref[e:e + 1])
        a = blend(lambda e: ah_ref[e:e + 1])
        t = jnp.where(t > 0, t, a * t)

        ws = blend(lambda e: ws_ref[e])
        t = jnp.dot(t, ws, preferred_element_type=jnp.float32)
        t = t + blend(lambda e: bs_ref[e:e + 1])
        a = blend(lambda e: as_ref[e:e + 1])
        t = jnp.where(t > 0, t, a * t)

        for m in range(3):
            pm = _im2col(t, 12, 1, qmod)
            wm = blend(lambda e: wm_ref[e, m])
            t = jnp.dot(pm, wm, preferred_element_type=jnp.float32)
            t = t + blend(lambda e: bm_ref[e, m:m + 1])
            a = blend(lambda e: am_ref[e, m:m + 1])
            t = jnp.where(t > 0, t, a * t)

        we = blend(lambda e: we_ref[e])
        t = jnp.dot(t, we, preferred_element_type=jnp.float32)
        t = t + blend(lambda e: be_ref[e:e + 1])
        a = blend(lambda e: ae_ref[e:e + 1])
        t = jnp.where(t > 0, t, a * t)

        pt = _im2col(t, 16, 1, qmod)
        wt = blend(lambda e: wt_ref[e])
        y = jnp.dot(pt, wt, preferred_element_type=jnp.float32)
        y = y + blend(lambda e: bt_ref[e:e + 1])
        out_ref[i] = y


def kernel(x, cls_w1, cls_b1, cls_w2, cls_b2, cls_w3, cls_b3, fc_w, fc_b,
           head_w, head_b, head_a, shrink_w, shrink_b, shrink_a,
           map_w, map_b, map_a, expand_w, expand_b, expand_a,
           tail_w, tail_b):
    B = x.shape[0]
    E, d, s = 3, 16, 12

    xflat = x.transpose(0, 2, 3, 1).reshape(B, 1024, 3)
    xp = (x.reshape(B, 3, 8, 4, 8, 4)
          .transpose(0, 2, 4, 1, 3, 5)
          .reshape(B, 64, 48))

    w1r = cls_w1.reshape(128, 48).T
    w2r = cls_w2.reshape(128, 128).T
    w3r = cls_w3.reshape(128, 128).T
    fcr = fc_w.T

    whr = head_w.transpose(0, 3, 4, 2, 1).reshape(E, 75, d)
    wsr = shrink_w.reshape(E, s, d).transpose(0, 2, 1)
    wmr = map_w.transpose(0, 1, 4, 5, 3, 2).reshape(E, 3, 108, s)
    wer = expand_w.reshape(E, d, s).transpose(0, 2, 1)

    ktab = np.array([[4 * (dd - 1) + 5 - r for r in range(4)]
                     for dd in range(3)])
    kvalid = (ktab >= 0) & (ktab <= 8)
    kc = np.clip(ktab, 0, 8)
    g = tail_w[:, :, :, kc, :][..., kc]
    vm = (kvalid[:, :, None, None] & kvalid[None, None, :, :])
    g = jnp.where(vm[None, None, None], g, 0.0)
    wtr = g.transpose(0, 3, 5, 2, 1, 4, 6).reshape(E, 144, 48)
    btr = jnp.broadcast_to(tail_b[:, :, None], (E, 3, 16)).reshape(E, 48)

    nb = _NB
    grid = (B // nb,)
    f32 = jnp.float32

    def bmap(i):
        return (i, 0, 0)

    def cmap2(i):
        return (0, 0)

    def cmap3(i):
        return (0, 0, 0)

    def cmap4(i):
        return (0, 0, 0, 0)

    c2 = lambda shp: pl.BlockSpec(shp, cmap2)
    c3 = lambda shp: pl.BlockSpec(shp, cmap3)
    c4 = lambda shp: pl.BlockSpec(shp, cmap4)

    y = pl.pallas_call(
        _fused_kernel,
        grid=grid,
        in_specs=[
            pl.BlockSpec((nb, 1024, 3), bmap),
            pl.BlockSpec((nb, 64, 48), bmap),
            c2((48, 128)), c2((1, 128)),
            c2((128, 128)), c2((1, 128)),
            c2((128, 128)), c2((1, 128)),
            c2((128, 3)), c2((1, 3)),
            c3((E, 75, d)), c2((E, d)), c2((E, d)),
            c3((E, d, s)), c2((E, s)), c2((E, s)),
            c4((E, 3, 108, s)), c3((E, 3, s)), c3((E, 3, s)),
            c3((E, s, d)), c2((E, d)), c2((E, d)),
            c3((E, 144, 48)), c2((E, 48)),
        ],
        out_specs=pl.BlockSpec((nb, 1024, 48), bmap),
        out_shape=jax.ShapeDtypeStruct((B, 1024, 48), f32),
    )(xflat, xp,
      w1r, cls_b1.reshape(1, 128), w2r, cls_b2.reshape(1, 128),
      w3r, cls_b3.reshape(1, 128), fcr, fc_b.reshape(1, 3),
      whr, head_b, head_a,
      wsr, shrink_b, shrink_a,
      wmr, map_b, map_a,
      wer, expand_b, expand_a,
      wtr, btr)

    sr = (y.reshape(B, 32, 32, 3, 4, 4)
          .transpose(0, 3, 1, 4, 2, 5)
          .reshape(B, 3, 128, 128))
    return sr


# speedup vs baseline: 5.4578x; 5.4578x over previous
import jax
import jax.numpy as jnp
import numpy as np
from jax.experimental import pallas as pl


_NB = 4


def _im2col_cm(t, khalf, qx, qy):
    C, L = t.shape
    pad = 32 * khalf + khalf
    tp = jnp.pad(t, ((0, 0), (pad, pad)))
    pieces = []
    for dy in range(-khalf, khalf + 1):
        for dx in range(-khalf, khalf + 1):
            off = pad + 32 * dy + dx
            p = jax.lax.slice(tp, (0, off), (C, off + L))
            m = ((qx + dx >= 0) & (qx + dx < 32)
                 & (qy + dy >= 0) & (qy + dy < 32))
            pieces.append(jnp.where(m, p, 0.0))
    return jnp.concatenate(pieces, axis=0)


def _fused_kernel(xcm_ref, xp_ref,
                  w1_ref, b1_ref, w2_ref, b2_ref, w3_ref, b3_ref,
                  fcw_ref, fcb_ref,
                  wh_ref, bh_ref, ah_ref,
                  ws_ref, bs_ref, as_ref,
                  wm_ref, bm_ref, am_ref,
                  we_ref, be_ref, ae_ref,
                  wt_ref, bt_ref,
                  out_ref):
    nb = out_ref.shape[0]
    L = nb * 1024

    xp = xp_ref[...].reshape(nb * 64, 48)
    h = xp @ w1_ref[...] + b1_ref[...]
    h = jnp.where(h > 0, h, 0.1 * h)
    h = h @ w2_ref[...] + b2_ref[...]
    h = jnp.where(h > 0, h, 0.1 * h)
    h = h @ w3_ref[...] + b3_ref[...]
    h = jnp.where(h > 0, h, 0.1 * h)
    gap = jnp.mean(h.reshape(nb, 64, 128), axis=1)
    logits = gap @ fcw_ref[...] + fcb_ref[...]

    l0 = logits[:, 0:1]
    l1 = logits[:, 1:2]
    l2 = logits[:, 2:3]
    e0 = (l0 >= l1) & (l0 >= l2)
    e1 = jnp.logical_not(e0) & (l1 >= l2)
    e2 = jnp.logical_not(e0) & jnp.logical_not(e1)
    coeff = jnp.concatenate(
        [e0.astype(jnp.float32), e1.astype(jnp.float32), e2.astype(jnp.float32)],
        axis=1)

    g = jax.lax.broadcasted_iota(jnp.int32, (1, L), 1)
    qx = g % 32
    qy = (g // 32) % 32

    def matmuls(P, wget, bget, aget, coeff):
        outs = []
        for i in range(nb):
            c0 = coeff[i:i + 1, 0:1]
            c1 = coeff[i:i + 1, 1:2]
            c2 = coeff[i:i + 1, 2:3]
            w = c0 * wget(0) + c1 * wget(1) + c2 * wget(2)
            Pi = jax.lax.slice(P, (0, i * 1024), (P.shape[0], (i + 1) * 1024))
            t = jnp.dot(w, Pi, preferred_element_type=jnp.float32)
            t = t + (c0 * bget(0) + c1 * bget(1) + c2 * bget(2))
            if aget is not None:
                a = c0 * aget(0) + c1 * aget(1) + c2 * aget(2)
                t = jnp.where(t > 0, t, a * t)
            outs.append(t)
        return jnp.concatenate(outs, axis=1)

    x8 = jnp.pad(xcm_ref[...], ((0, 5), (0, 0)))
    p0 = _im2col_cm(x8, 2, qx, qy)
    t = matmuls(p0, lambda e: wh_ref[e], lambda e: bh_ref[e],
                lambda e: ah_ref[e], coeff)

    t = matmuls(t, lambda e: ws_ref[e], lambda e: bs_ref[e],
                lambda e: as_ref[e], coeff)

    for m in range(3):
        pm = _im2col_cm(t, 1, qx, qy)
        t = matmuls(pm, lambda e: wm_ref[e, m], lambda e: bm_ref[e, m],
                    lambda e: am_ref[e, m], coeff)

    t = matmuls(t, lambda e: we_ref[e], lambda e: be_ref[e],
                lambda e: ae_ref[e], coeff)

    pt = _im2col_cm(t, 1, qx, qy)
    y = matmuls(pt, lambda e: wt_ref[e], lambda e: bt_ref[e], None, coeff)
    for i in range(nb):
        out_ref[i] = jax.lax.slice(y, (0, i * 1024), (48, (i + 1) * 1024))


def _pad_rows(w, rows):
    return jnp.pad(w, ((0, 0), (0, rows - w.shape[1]), (0, 0)))


def kernel(x, cls_w1, cls_b1, cls_w2, cls_b2, cls_w3, cls_b3, fc_w, fc_b,
           head_w, head_b, head_a, shrink_w, shrink_b, shrink_a,
           map_w, map_b, map_a, expand_w, expand_b, expand_a,
           tail_w, tail_b):
    B = x.shape[0]
    E, d, s = 3, 16, 12
    f32 = jnp.float32

    xcm = x.transpose(1, 0, 2, 3).reshape(3, B * 1024)
    xp = (x.reshape(B, 3, 8, 4, 8, 4)
          .transpose(0, 2, 4, 1, 3, 5)
          .reshape(B, 64, 48))

    w1r = cls_w1.reshape(128, 48).T
    w2r = cls_w2.reshape(128, 128).T
    w3r = cls_w3.reshape(128, 128).T
    fcr = fc_w.T

    whr = head_w.transpose(0, 1, 3, 4, 2)
    whr = jnp.pad(whr, ((0, 0),) * 4 + ((0, 5),)).reshape(E, d, 200)
    wsr = _pad_rows(shrink_w.reshape(E, s, d), d)
    bsr = jnp.pad(shrink_b, ((0, 0), (0, d - s)))[:, :, None]
    asr = jnp.pad(shrink_a, ((0, 0), (0, d - s)))[:, :, None]
    wmr = map_w.transpose(0, 1, 2, 4, 5, 3)
    wmr = jnp.pad(wmr, ((0, 0), (0, 0), (0, d - s), (0, 0), (0, 0), (0, d - s)))
    wmr = wmr.reshape(E, 3, d, 144)
    bmr = jnp.pad(map_b, ((0, 0), (0, 0), (0, d - s)))[:, :, :, None]
    amr = jnp.pad(map_a, ((0, 0), (0, 0), (0, d - s)))[:, :, :, None]
    wer = jnp.pad(expand_w.reshape(E, d, s), ((0, 0), (0, 0), (0, d - s)))

    ktab = np.array([[4 * (dd - 1) + 5 - r for r in range(4)]
                     for dd in range(3)])
    kvalid = (ktab >= 0) & (ktab <= 8)
    kc = np.clip(ktab, 0, 8)
    gw = tail_w[:, :, :, kc, :][..., kc]
    vm = (kvalid[:, :, None, None] & kvalid[None, None, :, :])
    gw = jnp.where(vm[None, None, None], gw, 0.0)
    wtr = gw.transpose(0, 1, 4, 6, 3, 5, 2).reshape(E, 48, 144)
    btr = jnp.broadcast_to(tail_b[:, :, None], (E, 3, 16)).reshape(E, 48, 1)

    nb = _NB
    grid = (B // nb,)

    def bmap(i):
        return (i, 0, 0)

    c2 = lambda shp: pl.BlockSpec(shp, lambda i: (0, 0))
    c3 = lambda shp: pl.BlockSpec(shp, lambda i: (0, 0, 0))
    c4 = lambda shp: pl.BlockSpec(shp, lambda i: (0, 0, 0, 0))

    y = pl.pallas_call(
        _fused_kernel,
        grid=grid,
        in_specs=[
            pl.BlockSpec((3, nb * 1024), lambda i: (0, i)),
            pl.BlockSpec((nb, 64, 48), bmap),
            c2((48, 128)), c2((1, 128)),
            c2((128, 128)), c2((1, 128)),
            c2((128, 128)), c2((1, 128)),
            c2((128, 3)), c2((1, 3)),
            c3((E, d, 200)), c3((E, d, 1)), c3((E, d, 1)),
            c3((E, d, d)), c3((E, d, 1)), c3((E, d, 1)),
            c4((E, 3, d, 144)), c4((E, 3, d, 1)), c4((E, 3, d, 1)),
            c3((E, d, d)), c3((E, d, 1)), c3((E, d, 1)),
            c3((E, 48, 144)), c3((E, 48, 1)),
        ],
        out_specs=pl.BlockSpec((nb, 48, 1024), bmap),
        out_shape=jax.ShapeDtypeStruct((B, 48, 1024), f32),
    )(xcm, xp,
      w1r, cls_b1.reshape(1, 128), w2r, cls_b2.reshape(1, 128),
      w3r, cls_b3.reshape(1, 128), fcr, fc_b.reshape(1, 3),
      whr, head_b[:, :, None], head_a[:, :, None],
      wsr, bsr, asr,
      wmr, bmr, amr,
      wer, expand_b[:, :, None], expand_a[:, :, None],
      wtr, btr)

    sr = (y.reshape(B, 3, 4, 4, 32, 32)
          .transpose(0, 1, 4, 2, 5, 3)
          .reshape(B, 3, 128, 128))
    return sr


# speedup vs baseline: 5.6548x; 1.0361x over previous
import jax
import jax.numpy as jnp
import numpy as np
from jax.experimental import pallas as pl


_NB = 4


def _make_masks(khalf, qx, qy):
    masks = {}
    for dy in range(-khalf, khalf + 1):
        for dx in range(-khalf, khalf + 1):
            if dy == 0 and dx == 0:
                masks[(dy, dx)] = None
                continue
            m = None
            if dx != 0:
                m = (qx + dx >= 0) & (qx + dx < 32)
            if dy != 0:
                my = (qy + dy >= 0) & (qy + dy < 32)
                m = my if m is None else (m & my)
            masks[(dy, dx)] = m
    return masks


def _im2col_cm(t, khalf, masks):
    C, L = t.shape
    pad = 32 * khalf + khalf
    tp = jnp.pad(t, ((0, 0), (pad, pad)))
    pieces = []
    for dy in range(-khalf, khalf + 1):
        for dx in range(-khalf, khalf + 1):
            off = pad + 32 * dy + dx
            p = jax.lax.slice(tp, (0, off), (C, off + L))
            m = masks[(dy, dx)]
            if m is not None:
                p = jnp.where(m, p, 0.0)
            pieces.append(p)
    return jnp.concatenate(pieces, axis=0)


def _fused_kernel(xcm_ref, xp_ref,
                  w1_ref, b1_ref, w2_ref, b2_ref, w3_ref, b3_ref,
                  fcw_ref, fcb_ref,
                  wh_ref, bh_ref, ah_ref,
                  ws_ref, bs_ref, as_ref,
                  wm_ref, bm_ref, am_ref,
                  we_ref, be_ref, ae_ref,
                  wt_ref, bt_ref,
                  out_ref):
    nb = out_ref.shape[0]

    xp = xp_ref[...].reshape(nb * 64, 48)
    h = xp @ w1_ref[...] + b1_ref[...]
    h = jnp.where(h > 0, h, 0.1 * h)
    h = h @ w2_ref[...] + b2_ref[...]
    h = jnp.where(h > 0, h, 0.1 * h)
    h = h @ w3_ref[...] + b3_ref[...]
    h = jnp.where(h > 0, h, 0.1 * h)
    gap = jnp.mean(h.reshape(nb, 64, 128), axis=1)
    logits = gap @ fcw_ref[...] + fcb_ref[...]

    l0 = logits[:, 0:1]
    l1 = logits[:, 1:2]
    l2 = logits[:, 2:3]
    e0 = (l0 >= l1) & (l0 >= l2)
    e1 = jnp.logical_not(e0) & (l1 >= l2)
    e2 = jnp.logical_not(e0) & jnp.logical_not(e1)
    coeff = jnp.concatenate(
        [e0.astype(jnp.float32), e1.astype(jnp.float32), e2.astype(jnp.float32)],
        axis=1)

    L = nb * 1024
    g = jax.lax.broadcasted_iota(jnp.int32, (1, L), 1)
    qx = g % 32
    qy = (g // 32) % 32
    masks1 = _make_masks(1, qx, qy)
    masks2 = _make_masks(2, qx, qy)

    def matmuls(P, wget, bget, aget):
        outs = []
        for i in range(nb):
            c0 = coeff[i:i + 1, 0:1]
            c1 = coeff[i:i + 1, 1:2]
            c2 = coeff[i:i + 1, 2:3]
            w = c0 * wget(0) + c1 * wget(1) + c2 * wget(2)
            Pi = jax.lax.slice(P, (0, i * 1024), (P.shape[0], (i + 1) * 1024))
            t = jnp.dot(w, Pi, preferred_element_type=jnp.float32)
            t = t + (c0 * bget(0) + c1 * bget(1) + c2 * bget(2))
            if aget is not None:
                a = c0 * aget(0) + c1 * aget(1) + c2 * aget(2)
                t = jnp.where(t > 0, t, a * t)
            outs.append(t)
        return jnp.concatenate(outs, axis=1)

    x8 = jnp.pad(xcm_ref[...], ((0, 5), (0, 0)))
    p0 = _im2col_cm(x8, 2, masks2)
    t = matmuls(p0, lambda e: wh_ref[e], lambda e: bh_ref[e],
                lambda e: ah_ref[e])

    t = matmuls(t, lambda e: ws_ref[e], lambda e: bs_ref[e],
                lambda e: as_ref[e])

    for m in range(3):
        pm = _im2col_cm(t, 1, masks1)
        t = matmuls(pm, lambda e: wm_ref[e, m], lambda e: bm_ref[e, m],
                    lambda e: am_ref[e, m])

    t = matmuls(t, lambda e: we_ref[e], lambda e: be_ref[e],
                lambda e: ae_ref[e])

    pt = _im2col_cm(t, 1, masks1)
    y = matmuls(pt, lambda e: wt_ref[e], lambda e: bt_ref[e], None)
    for i in range(nb):
        out_ref[i] = jax.lax.slice(y, (0, i * 1024), (48, (i + 1) * 1024))


def kernel(x, cls_w1, cls_b1, cls_w2, cls_b2, cls_w3, cls_b3, fc_w, fc_b,
           head_w, head_b, head_a, shrink_w, shrink_b, shrink_a,
           map_w, map_b, map_a, expand_w, expand_b, expand_a,
           tail_w, tail_b):
    B = x.shape[0]
    E, d, s = 3, 16, 12
    f32 = jnp.float32

    xcm = x.transpose(1, 0, 2, 3).reshape(3, B * 1024)
    xp = (x.reshape(B, 3, 8, 4, 8, 4)
          .transpose(0, 2, 4, 1, 3, 5)
          .reshape(B, 64, 48))

    w1r = cls_w1.reshape(128, 48).T
    w2r = cls_w2.reshape(128, 128).T
    w3r = cls_w3.reshape(128, 128).T
    fcr = fc_w.T

    whr = head_w.transpose(0, 1, 3, 4, 2)
    whr = jnp.pad(whr, ((0, 0),) * 4 + ((0, 5),)).reshape(E, d, 200)
    wsr = jnp.pad(shrink_w.reshape(E, s, d), ((0, 0), (0, d - s), (0, 0)))
    bsr = jnp.pad(shrink_b, ((0, 0), (0, d - s)))[:, :, None]
    asr = jnp.pad(shrink_a, ((0, 0), (0, d - s)))[:, :, None]
    wmr = map_w.transpose(0, 1, 2, 4, 5, 3)
    wmr = jnp.pad(wmr, ((0, 0), (0, 0), (0, d - s), (0, 0), (0, 0), (0, d - s)))
    wmr = wmr.reshape(E, 3, d, 144)
    bmr = jnp.pad(map_b, ((0, 0), (0, 0), (0, d - s)))[:, :, :, None]
    amr = jnp.pad(map_a, ((0, 0), (0, 0), (0, d - s)))[:, :, :, None]
    wer = jnp.pad(expand_w.reshape(E, d, s), ((0, 0), (0, 0), (0, d - s)))

    ktab = np.array([[4 * (dd - 1) + 5 - r for r in range(4)]
                     for dd in range(3)])
    kvalid = (ktab >= 0) & (ktab <= 8)
    kc = np.clip(ktab, 0, 8)
    gw = tail_w[:, :, :, kc, :][..., kc]
    vm = (kvalid[:, :, None, None] & kvalid[None, None, :, :])
    gw = jnp.where(vm[None, None, None], gw, 0.0)
    wtr = gw.transpose(0, 1, 4, 6, 3, 5, 2).reshape(E, 48, 144)
    btr = jnp.broadcast_to(tail_b[:, :, None], (E, 3, 16)).reshape(E, 48, 1)

    nb = _NB
    grid = (B // nb,)

    def bmap(i):
        return (i, 0, 0)

    c2 = lambda shp: pl.BlockSpec(shp, lambda i: (0, 0))
    c3 = lambda shp: pl.BlockSpec(shp, lambda i: (0, 0, 0))
    c4 = lambda shp: pl.BlockSpec(shp, lambda i: (0, 0, 0, 0))

    y = pl.pallas_call(
        _fused_kernel,
        grid=grid,
        in_specs=[
            pl.BlockSpec((3, nb * 1024), lambda i: (0, i)),
            pl.BlockSpec((nb, 64, 48), bmap),
            c2((48, 128)), c2((1, 128)),
            c2((128, 128)), c2((1, 128)),
            c2((128, 128)), c2((1, 128)),
            c2((128, 3)), c2((1, 3)),
            c3((E, d, 200)), c3((E, d, 1)), c3((E, d, 1)),
            c3((E, d, d)), c3((E, d, 1)), c3((E, d, 1)),
            c4((E, 3, d, 144)), c4((E, 3, d, 1)), c4((E, 3, d, 1)),
            c3((E, d, d)), c3((E, d, 1)), c3((E, d, 1)),
            c3((E, 48, 144)), c3((E, 48, 1)),
        ],
        out_specs=pl.BlockSpec((nb, 48, 1024), bmap),
        out_shape=jax.ShapeDtypeStruct((B, 48, 1024), f32),
    )(xcm, xp,
      w1r, cls_b1.reshape(1, 128), w2r, cls_b2.reshape(1, 128),
      w3r, cls_b3.reshape(1, 128), fcr, fc_b.reshape(1, 3),
      whr, head_b[:, :, None], head_a[:, :, None],
      wsr, bsr, asr,
      wmr, bmr, amr,
      wer, expand_b[:, :, None], expand_a[:, :, None],
      wtr, btr)

    sr = (y.reshape(B, 3, 4, 4, 32, 32)
          .transpose(0, 1, 4, 2, 5, 3)
          .reshape(B, 3, 128, 128))
    return sr


# speedup vs baseline: 5.6868x; 1.0057x over previous
import jax
import jax.numpy as jnp
import numpy as np
from jax.experimental import pallas as pl
from jax.experimental.pallas import tpu as pltpu


_NB = 4


def _make_masks(khalf, qx, qy):
    masks = {}
    for dy in range(-khalf, khalf + 1):
        for dx in range(-khalf, khalf + 1):
            if dy == 0 and dx == 0:
                masks[(dy, dx)] = None
                continue
            m = None
            if dx != 0:
                m = (qx + dx >= 0) & (qx + dx < 32)
            if dy != 0:
                my = (qy + dy >= 0) & (qy + dy < 32)
                m = my if m is None else (m & my)
            masks[(dy, dx)] = m
    return masks


def _im2col_cm(t, khalf, masks, p_ref):
    C, L = t.shape
    pad = 32 * khalf + khalf
    tp = jnp.pad(t, ((0, 0), (pad, pad)))
    k = 0
    for dy in range(-khalf, khalf + 1):
        for dx in range(-khalf, khalf + 1):
            off = pad + 32 * dy + dx
            p = jax.lax.slice(tp, (0, off), (C, off + L))
            m = masks[(dy, dx)]
            if m is not None:
                p = jnp.where(m, p, 0.0)
            p_ref[k * C:(k + 1) * C, :] = p
            k += 1
    return p_ref[...]


def _fused_kernel(xcm_ref, xp_ref,
                  w1_ref, b1_ref, w2_ref, b2_ref, w3_ref, b3_ref,
                  fcw_ref, fcb_ref,
                  wh_ref, bh_ref, ah_ref,
                  ws_ref, bs_ref, as_ref,
                  wm_ref, bm_ref, am_ref,
                  we_ref, be_ref, ae_ref,
                  wt_ref, bt_ref,
                  out_ref, ph_ref, pm_ref):
    nb = out_ref.shape[0]

    xp = xp_ref[...].reshape(nb * 64, 48)
    h = xp @ w1_ref[...] + b1_ref[...]
    h = jnp.where(h > 0, h, 0.1 * h)
    h = h @ w2_ref[...] + b2_ref[...]
    h = jnp.where(h > 0, h, 0.1 * h)
    h = h @ w3_ref[...] + b3_ref[...]
    h = jnp.where(h > 0, h, 0.1 * h)
    gap = jnp.mean(h.reshape(nb, 64, 128), axis=1)
    logits = gap @ fcw_ref[...] + fcb_ref[...]

    l0 = logits[:, 0:1]
    l1 = logits[:, 1:2]
    l2 = logits[:, 2:3]
    e0 = (l0 >= l1) & (l0 >= l2)
    e1 = jnp.logical_not(e0) & (l1 >= l2)
    e2 = jnp.logical_not(e0) & jnp.logical_not(e1)
    coeff = jnp.concatenate(
        [e0.astype(jnp.float32), e1.astype(jnp.float32), e2.astype(jnp.float32)],
        axis=1)

    L = nb * 1024
    g = jax.lax.broadcasted_iota(jnp.int32, (1, L), 1)
    qx = g % 32
    qy = (g // 32) % 32
    masks1 = _make_masks(1, qx, qy)
    masks2 = _make_masks(2, qx, qy)

    def matmuls(P, wget, bget, aget):
        outs = []
        for i in range(nb):
            c0 = coeff[i:i + 1, 0:1]
            c1 = coeff[i:i + 1, 1:2]
            c2 = coeff[i:i + 1, 2:3]
            w = c0 * wget(0) + c1 * wget(1) + c2 * wget(2)
            Pi = jax.lax.slice(P, (0, i * 1024), (P.shape[0], (i + 1) * 1024))
            t = jnp.dot(w, Pi, preferred_element_type=jnp.float32)
            t = t + (c0 * bget(0) + c1 * bget(1) + c2 * bget(2))
            if aget is not None:
                a = c0 * aget(0) + c1 * aget(1) + c2 * aget(2)
                t = jnp.where(t > 0, t, a * t)
            outs.append(t)
        return jnp.concatenate(outs, axis=1)

    x8 = jnp.pad(xcm_ref[...], ((0, 5), (0, 0)))
    p0 = _im2col_cm(x8, 2, masks2, ph_ref)
    t = matmuls(p0, lambda e: wh_ref[e], lambda e: bh_ref[e],
                lambda e: ah_ref[e])

    t = matmuls(t, lambda e: ws_ref[e], lambda e: bs_ref[e],
                lambda e: as_ref[e])

    for m in range(3):
        pm = _im2col_cm(t, 1, masks1, pm_ref)
        t = matmuls(pm, lambda e: wm_ref[e, m], lambda e: bm_ref[e, m],
                    lambda e: am_ref[e, m])

    t = matmuls(t, lambda e: we_ref[e], lambda e: be_ref[e],
                lambda e: ae_ref[e])

    pt = _im2col_cm(t, 1, masks1, pm_ref)
    y = matmuls(pt, lambda e: wt_ref[e], lambda e: bt_ref[e], None)
    for i in range(nb):
        out_ref[i] = jax.lax.slice(y, (0, i * 1024), (48, (i + 1) * 1024))


def kernel(x, cls_w1, cls_b1, cls_w2, cls_b2, cls_w3, cls_b3, fc_w, fc_b,
           head_w, head_b, head_a, shrink_w, shrink_b, shrink_a,
           map_w, map_b, map_a, expand_w, expand_b, expand_a,
           tail_w, tail_b):
    B = x.shape[0]
    E, d, s = 3, 16, 12
    f32 = jnp.float32

    xcm = x.transpose(1, 0, 2, 3).reshape(3, B * 1024)
    xp = (x.reshape(B, 3, 8, 4, 8, 4)
          .transpose(0, 2, 4, 1, 3, 5)
          .reshape(B, 64, 48))

    w1r = cls_w1.reshape(128, 48).T
    w2r = cls_w2.reshape(128, 128).T
    w3r = cls_w3.reshape(128, 128).T
    fcr = fc_w.T

    whr = head_w.transpose(0, 1, 3, 4, 2)
    whr = jnp.pad(whr, ((0, 0),) * 4 + ((0, 5),)).reshape(E, d, 200)
    wsr = jnp.pad(shrink_w.reshape(E, s, d), ((0, 0), (0, d - s), (0, 0)))
    bsr = jnp.pad(shrink_b, ((0, 0), (0, d - s)))[:, :, None]
    asr = jnp.pad(shrink_a, ((0, 0), (0, d - s)))[:, :, None]
    wmr = map_w.transpose(0, 1, 2, 4, 5, 3)
    wmr = jnp.pad(wmr, ((0, 0), (0, 0), (0, d - s), (0, 0), (0, 0), (0, d - s)))
    wmr = wmr.reshape(E, 3, d, 144)
    bmr = jnp.pad(map_b, ((0, 0), (0, 0), (0, d - s)))[:, :, :, None]
    amr = jnp.pad(map_a, ((0, 0), (0, 0), (0, d - s)))[:, :, :, None]
    wer = jnp.pad(expand_w.reshape(E, d, s), ((0, 0), (0, 0), (0, d - s)))

    ktab = np.array([[4 * (dd - 1) + 5 - r for r in range(4)]
                     for dd in range(3)])
    kvalid = (ktab >= 0) & (ktab <= 8)
    kc = np.clip(ktab, 0, 8)
    gw = tail_w[:, :, :, kc, :][..., kc]
    vm = (kvalid[:, :, None, None] & kvalid[None, None, :, :])
    gw = jnp.where(vm[None, None, None], gw, 0.0)
    wtr = gw.transpose(0, 1, 4, 6, 3, 5, 2).reshape(E, 48, 144)
    btr = jnp.broadcast_to(tail_b[:, :, None], (E, 3, 16)).reshape(E, 48, 1)

    nb = _NB
    grid = (B // nb,)

    def bmap(i):
        return (i, 0, 0)

    c2 = lambda shp: pl.BlockSpec(shp, lambda i: (0, 0))
    c3 = lambda shp: pl.BlockSpec(shp, lambda i: (0, 0, 0))
    c4 = lambda shp: pl.BlockSpec(shp, lambda i: (0, 0, 0, 0))

    y = pl.pallas_call(
        _fused_kernel,
        grid=grid,
        in_specs=[
            pl.BlockSpec((3, nb * 1024), lambda i: (0, i)),
            pl.BlockSpec((nb, 64, 48), bmap),
            c2((48, 128)), c2((1, 128)),
            c2((128, 128)), c2((1, 128)),
            c2((128, 128)), c2((1, 128)),
            c2((128, 3)), c2((1, 3)),
            c3((E, d, 200)), c3((E, d, 1)), c3((E, d, 1)),
            c3((E, d, d)), c3((E, d, 1)), c3((E, d, 1)),
            c4((E, 3, d, 144)), c4((E, 3, d, 1)), c4((E, 3, d, 1)),
            c3((E, d, d)), c3((E, d, 1)), c3((E, d, 1)),
            c3((E, 48, 144)), c3((E, 48, 1)),
        ],
        out_specs=pl.BlockSpec((nb, 48, 1024), bmap),
        out_shape=jax.ShapeDtypeStruct((B, 48, 1024), f32),
        scratch_shapes=[
            pltpu.VMEM((200, nb * 1024), f32),
            pltpu.VMEM((144, nb * 1024), f32),
        ],
    )(xcm, xp,
      w1r, cls_b1.reshape(1, 128), w2r, cls_b2.reshape(1, 128),
      w3r, cls_b3.reshape(1, 128), fcr, fc_b.reshape(1, 3),
      whr, head_b[:, :, None], head_a[:, :, None],
      wsr, bsr, asr,
      wmr, bmr, amr,
      wer, expand_b[:, :, None], expand_a[:, :, None],
      wtr, btr)

    sr = (y.reshape(B, 3, 4, 4, 32, 32)
          .transpose(0, 1, 4, 2, 5, 3)
          .reshape(B, 3, 128, 128))
    return sr


# speedup vs baseline: 5.8056x; 1.0209x over previous
import jax
import jax.numpy as jnp
import numpy as np
from jax.experimental import pallas as pl
from jax.experimental.pallas import tpu as pltpu


_NB = 4


def _make_masks(khalf, qx, qy):
    masks = {}
    for dy in range(-khalf, khalf + 1):
        for dx in range(-khalf, khalf + 1):
            if dy == 0 and dx == 0:
                masks[(dy, dx)] = None
                continue
            m = None
            if dx != 0:
                m = (qx + dx >= 0) & (qx + dx < 32)
            if dy != 0:
                my = (qy + dy >= 0) & (qy + dy < 32)
                m = my if m is None else (m & my)
            masks[(dy, dx)] = m
    return masks


def _im2col_cm(t, khalf, masks, p_ref):
    C, L = t.shape
    pad = 32 * khalf + khalf
    tp = jnp.pad(t, ((0, 0), (pad, pad)))
    k = 0
    for dy in range(-khalf, khalf + 1):
        for dx in range(-khalf, khalf + 1):
            off = pad + 32 * dy + dx
            p = jax.lax.slice(tp, (0, off), (C, off + L))
            m = masks[(dy, dx)]
            if m is not None:
                p = jnp.where(m, p, 0.0)
            p_ref[k * C:(k + 1) * C, :] = p
            k += 1
    return p_ref[...]


def _fused_kernel(xcm_ref, xp_ref,
                  w1_ref, b1_ref, w2_ref, b2_ref, w3_ref, b3_ref,
                  fcw_ref, fcb_ref,
                  wh_ref, bh_ref, ah_ref,
                  ws_ref, bs_ref, as_ref,
                  wm_ref, bm_ref, am_ref,
                  we_ref, be_ref, ae_ref,
                  wt_ref, bt_ref,
                  out_ref, ph_ref, pm_ref):
    nb = out_ref.shape[0]

    xp = xp_ref[...].reshape(nb * 64, 48)
    h = xp @ w1_ref[...] + b1_ref[...]
    h = jnp.where(h > 0, h, 0.1 * h)
    h = h @ w2_ref[...] + b2_ref[...]
    h = jnp.where(h > 0, h, 0.1 * h)
    h = h @ w3_ref[...] + b3_ref[...]
    h = jnp.where(h > 0, h, 0.1 * h)
    gap = jnp.mean(h.reshape(nb, 64, 128), axis=1)
    logits = gap @ fcw_ref[...] + fcb_ref[...]

    l0 = logits[:, 0:1]
    l1 = logits[:, 1:2]
    l2 = logits[:, 2:3]
    e0 = (l0 >= l1) & (l0 >= l2)
    e1 = jnp.logical_not(e0) & (l1 >= l2)
    e2 = jnp.logical_not(e0) & jnp.logical_not(e1)
    coeff = jnp.concatenate(
        [e0.astype(jnp.float32), e1.astype(jnp.float32), e2.astype(jnp.float32)],
        axis=1)

    L = nb * 1024
    g = jax.lax.broadcasted_iota(jnp.int32, (1, L), 1)
    qx = g % 32
    qy = (g // 32) % 32
    masks1 = _make_masks(1, qx, qy)
    masks2 = _make_masks(2, qx, qy)

    def matmuls(P, wget, bget, aget):
        outs = []
        for i in range(nb):
            c0 = coeff[i:i + 1, 0:1]
            c1 = coeff[i:i + 1, 1:2]
            c2 = coeff[i:i + 1, 2:3]
            w = c0 * wget(0) + c1 * wget(1) + c2 * wget(2)
            Pi = jax.lax.slice(P, (0, i * 1024), (P.shape[0], (i + 1) * 1024))
            t = jnp.dot(w, Pi, preferred_element_type=jnp.float32)
            t = t + (c0 * bget(0) + c1 * bget(1) + c2 * bget(2))
            if aget is not None:
                a = c0 * aget(0) + c1 * aget(1) + c2 * aget(2)
                t = jnp.where(t > 0, t, a * t)
            outs.append(t)
        return jnp.concatenate(outs, axis=1)

    x8 = jnp.pad(xcm_ref[...], ((0, 5), (0, 0)))
    p0 = _im2col_cm(x8, 2, masks2, ph_ref)
    t = matmuls(p0, lambda e: wh_ref[e], lambda e: bh_ref[e],
                lambda e: ah_ref[e])

    t = matmuls(t, lambda e: ws_ref[e], lambda e: bs_ref[e],
                lambda e: as_ref[e])

    for m in range(3):
        pm = _im2col_cm(t, 1, masks1, pm_ref)
        t = matmuls(pm, lambda e: wm_ref[e, m], lambda e: bm_ref[e, m],
                    lambda e: am_ref[e, m])

    t = matmuls(t, lambda e: we_ref[e], lambda e: be_ref[e],
                lambda e: ae_ref[e])

    pt = _im2col_cm(t, 1, masks1, pm_ref)
    y = matmuls(pt, lambda e: wt_ref[e], lambda e: bt_ref[e], None)
    yb = y.astype(jnp.bfloat16)
    for i in range(nb):
        out_ref[i] = jax.lax.slice(yb, (0, i * 1024), (48, (i + 1) * 1024))


def kernel(x, cls_w1, cls_b1, cls_w2, cls_b2, cls_w3, cls_b3, fc_w, fc_b,
           head_w, head_b, head_a, shrink_w, shrink_b, shrink_a,
           map_w, map_b, map_a, expand_w, expand_b, expand_a,
           tail_w, tail_b):
    B = x.shape[0]
    E, d, s = 3, 16, 12
    f32 = jnp.float32

    xcm = x.transpose(1, 0, 2, 3).reshape(3, B * 1024)
    xp = (x.reshape(B, 3, 8, 4, 8, 4)
          .transpose(0, 2, 4, 1, 3, 5)
          .reshape(B, 64, 48))

    w1r = cls_w1.reshape(128, 48).T
    w2r = cls_w2.reshape(128, 128).T
    w3r = cls_w3.reshape(128, 128).T
    fcr = fc_w.T

    whr = head_w.transpose(0, 1, 3, 4, 2)
    whr = jnp.pad(whr, ((0, 0),) * 4 + ((0, 5),)).reshape(E, d, 200)
    wsr = jnp.pad(shrink_w.reshape(E, s, d), ((0, 0), (0, d - s), (0, 0)))
    bsr = jnp.pad(shrink_b, ((0, 0), (0, d - s)))[:, :, None]
    asr = jnp.pad(shrink_a, ((0, 0), (0, d - s)))[:, :, None]
    wmr = map_w.transpose(0, 1, 2, 4, 5, 3)
    wmr = jnp.pad(wmr, ((0, 0), (0, 0), (0, d - s), (0, 0), (0, 0), (0, d - s)))
    wmr = wmr.reshape(E, 3, d, 144)
    bmr = jnp.pad(map_b, ((0, 0), (0, 0), (0, d - s)))[:, :, :, None]
    amr = jnp.pad(map_a, ((0, 0), (0, 0), (0, d - s)))[:, :, :, None]
    wer = jnp.pad(expand_w.reshape(E, d, s), ((0, 0), (0, 0), (0, d - s)))

    ktab = np.array([[4 * (dd - 1) + 5 - r for r in range(4)]
                     for dd in range(3)])
    kvalid = (ktab >= 0) & (ktab <= 8)
    kc = np.clip(ktab, 0, 8)
    gw = tail_w[:, :, :, kc, :][..., kc]
    vm = (kvalid[:, :, None, None] & kvalid[None, None, :, :])
    gw = jnp.where(vm[None, None, None], gw, 0.0)
    wtr = gw.transpose(0, 1, 4, 6, 3, 5, 2).reshape(E, 48, 144)
    btr = jnp.broadcast_to(tail_b[:, :, None], (E, 3, 16)).reshape(E, 48, 1)

    nb = _NB
    grid = (B // nb,)

    def bmap(i):
        return (i, 0, 0)

    c2 = lambda shp: pl.BlockSpec(shp, lambda i: (0, 0))
    c3 = lambda shp: pl.BlockSpec(shp, lambda i: (0, 0, 0))
    c4 = lambda shp: pl.BlockSpec(shp, lambda i: (0, 0, 0, 0))

    y = pl.pallas_call(
        _fused_kernel,
        grid=grid,
        in_specs=[
            pl.BlockSpec((3, nb * 1024), lambda i: (0, i)),
            pl.BlockSpec((nb, 64, 48), bmap),
            c2((48, 128)), c2((1, 128)),
            c2((128, 128)), c2((1, 128)),
            c2((128, 128)), c2((1, 128)),
            c2((128, 3)), c2((1, 3)),
            c3((E, d, 200)), c3((E, d, 1)), c3((E, d, 1)),
            c3((E, d, d)), c3((E, d, 1)), c3((E, d, 1)),
            c4((E, 3, d, 144)), c4((E, 3, d, 1)), c4((E, 3, d, 1)),
            c3((E, d, d)), c3((E, d, 1)), c3((E, d, 1)),
            c3((E, 48, 144)), c3((E, 48, 1)),
        ],
        out_specs=pl.BlockSpec((nb, 48, 1024), bmap),
        out_shape=jax.ShapeDtypeStruct((B, 48, 1024), jnp.bfloat16),
        scratch_shapes=[
            pltpu.VMEM((200, nb * 1024), f32),
            pltpu.VMEM((144, nb * 1024), f32),
        ],
    )(xcm, xp,
      w1r, cls_b1.reshape(1, 128), w2r, cls_b2.reshape(1, 128),
      w3r, cls_b3.reshape(1, 128), fcr, fc_b.reshape(1, 3),
      whr, head_b[:, :, None], head_a[:, :, None],
      wsr, bsr, asr,
      wmr, bmr, amr,
      wer, expand_b[:, :, None], expand_a[:, :, None],
      wtr, btr)

    sr = (y.reshape(B, 3, 4, 4, 32, 32)
          .transpose(0, 1, 4, 2, 5, 3)
          .reshape(B, 3, 128, 128)
          .astype(f32))
    return sr
